# baseline (speedup 1.0000x reference)
import functools

import jax
import jax.numpy as jnp
from jax import lax
from jax.experimental import pallas as pl
from jax.experimental.pallas import tpu as pltpu
from jax.experimental.pallas import tpu_sc as plsc

N_NODES = 10000
N_EDGES = 320000
D = 128
LANES = 16
NCORES = 2
NSUB = 16
NBUF = 4
CHUNK = 128
NCHUNK = 160
NTRIP = NCHUNK // NBUF
EPT = NCHUNK * CHUNK
E_PAD = NSUB * EPT
NPC = N_NODES // NCORES
ACC_ROWS = NPC + 8
ROWS_PT = 312
TAIL_ROWS = NPC - NSUB * ROWS_PT
ZTAIL = ACC_ROWS - NSUB * ROWS_PT
ZROWS = 8

_sc_mesh = plsc.VectorSubcoreMesh(core_axis_name="c", subcore_axis_name="s")


@functools.partial(
    pl.kernel,
    mesh=_sc_mesh,
    out_type=jax.ShapeDtypeStruct((N_NODES, D), jnp.float32),
    scratch_types=[
        pltpu.VMEM((NBUF, 2, CHUNK), jnp.int32),
        pltpu.VMEM((NBUF, CHUNK), jnp.float32),
        pltpu.VMEM((CHUNK, D), jnp.float32),
        pltpu.VMEM((CHUNK, D), jnp.float32),
        pltpu.VMEM((CHUNK, D), jnp.float32),
        pltpu.VMEM((CHUNK, D), jnp.float32),
        pltpu.VMEM((ZROWS, D), jnp.float32),
        pltpu.VMEM_SHARED((ACC_ROWS, D), jnp.float32),
        pltpu.SemaphoreType.DMA,
        pltpu.SemaphoreType.DMA,
        pltpu.SemaphoreType.DMA,
        pltpu.SemaphoreType.DMA,
        pltpu.SemaphoreType.DMA,
        pltpu.SemaphoreType.DMA,
        pltpu.SemaphoreType.DMA,
        pltpu.SemaphoreType.DMA,
        pltpu.SemaphoreType.DMA,
        pltpu.SemaphoreType.DMA,
        pltpu.SemaphoreType.DMA,
        pltpu.SemaphoreType.DMA,
        pltpu.SemaphoreType.DMA,
        pltpu.SemaphoreType.DMA,
        pltpu.SemaphoreType.DMA,
        pltpu.SemaphoreType.DMA,
    ],
)
def _segment_sum(x_hbm, edges_hbm, w_hbm, out_hbm, ebuf, wbuf, r0, r1, r2, r3,
                 zero_v, acc_sh,
                 is0, is1, is2, is3, ws0, ws1, ws2, ws3,
                 gs0, gs1, gs2, gs3, ss0, ss1, ss2, ss3):
    core = lax.axis_index("c")
    sub = lax.axis_index("s")
    rows = [r0, r1, r2, r3]
    isem = [is0, is1, is2, is3]
    wsem = [ws0, ws1, ws2, ws3]
    gsem = [gs0, gs1, gs2, gs3]
    ssem = [ss0, ss1, ss2, ss3]

    base = jnp.full((LANES,), core * NPC, jnp.int32)
    npc = jnp.full((LANES,), NPC, jnp.int32)
    izero = jnp.zeros((LANES,), jnp.int32)

    def idx_start(j, b):
        row = sub * NCHUNK + j
        pltpu.async_copy(edges_hbm.at[row], ebuf.at[b], isem[b])
        pltpu.async_copy(w_hbm.at[row], wbuf.at[b], wsem[b])

    def idx_wait(j, b):
        row = sub * NCHUNK + j
        pltpu.make_async_copy(edges_hbm.at[row], ebuf.at[b],
                              isem[b]).wait()
        pltpu.make_async_copy(w_hbm.at[row], wbuf.at[b], wsem[b]).wait()

    def redirect(b):
        for g in range(CHUNK // LANES):
            sl = pl.ds(g * LANES, LANES)
            t = ebuf[b, 1, sl] - base
            keep = (t >= izero) & (t < npc)
            ebuf[b, 1, sl] = jnp.where(keep, t, npc)

    def gather_start(b):
        pltpu.async_copy(x_hbm.at[ebuf.at[b, 0]], rows[b], gsem[b])

    def gather_wait(b):
        pltpu.make_async_copy(x_hbm.at[ebuf.at[b, 0]], rows[b],
                              gsem[b]).wait()

    def weight(b):
        rv = rows[b]

        def wbody(g, carry):
            wvec = wbuf[b, pl.ds(g * LANES, LANES)]
            for e in range(LANES):
                wb = wvec[e]
                row = g * LANES + e
                for cg in range(D // LANES):
                    sl = pl.ds(cg * LANES, LANES)
                    rv[row, sl] = rv[row, sl] * wb
            return carry

        lax.fori_loop(0, CHUNK // LANES, wbody, 0)

    def scatter_start(b):
        pltpu.async_copy(rows[b], acc_sh.at[ebuf.at[b, 1]], ssem[b],
                         add=True)

    def scatter_wait(b):
        pltpu.make_async_copy(rows[b], acc_sh.at[ebuf.at[b, 1]],
                              ssem[b]).wait()

    zvec = jnp.zeros((LANES,), jnp.float32)
    for r in range(ZROWS):
        for cg in range(D // LANES):
            zero_v[r, pl.ds(cg * LANES, LANES)] = zvec

    def zbody(i, carry):
        off = pl.multiple_of(sub * ROWS_PT + i * ZROWS, 8)
        pltpu.sync_copy(zero_v, acc_sh.at[pl.ds(off, ZROWS)])
        return carry

    lax.fori_loop(0, ROWS_PT // ZROWS, zbody, 0)

    @pl.when(sub == 0)
    def _zero_tail():
        pltpu.sync_copy(zero_v, acc_sh.at[pl.ds(NSUB * ROWS_PT, ZROWS)])
        pltpu.sync_copy(zero_v,
                        acc_sh.at[pl.ds(NSUB * ROWS_PT + ZROWS, ZROWS)])

    plsc.subcore_barrier()

    idx_start(0, 0)
    idx_start(1, 1)
    idx_wait(0, 0)
    redirect(0)
    gather_start(0)

    def body(i, carry):
        for b in range(NBUF):
            j = i * NBUF + b
            bd = (b + 2) % NBUF
            bn = (b + 1) % NBUF
            if b < 2:
                @pl.when(i > 0)
                def _drain():
                    scatter_wait(bd)
            else:
                scatter_wait(bd)
            if b < 2:
                idx_start(j + 2, bd)
            else:
                @pl.when(i < NTRIP - 1)
                def _pref_idx():
                    idx_start(j + 2, bd)
            if b < 3:
                idx_wait(j + 1, bn)
                redirect(bn)
                gather_start(bn)
            else:
                @pl.when(i < NTRIP - 1)
                def _pref_gather():
                    idx_wait(j + 1, bn)
                    redirect(bn)
                    gather_start(bn)
            gather_wait(b)
            weight(b)
            scatter_start(b)
        return carry

    lax.fori_loop(0, NTRIP, body, 0)
    scatter_wait((NCHUNK - 2) % NBUF)
    scatter_wait((NCHUNK - 1) % NBUF)
    plsc.subcore_barrier()

    aoff = pl.multiple_of(sub * ROWS_PT, 8)
    ooff = pl.multiple_of(core * NPC + sub * ROWS_PT, 8)
    pltpu.sync_copy(acc_sh.at[pl.ds(aoff, ROWS_PT)],
                    out_hbm.at[pl.ds(ooff, ROWS_PT)])

    @pl.when(sub == 0)
    def _write_tail():
        toff = pl.multiple_of(core * NPC + NSUB * ROWS_PT, 8)
        pltpu.sync_copy(acc_sh.at[pl.ds(NSUB * ROWS_PT, TAIL_ROWS)],
                        out_hbm.at[pl.ds(toff, TAIL_ROWS)])


def _mlp_body(x_ref, agg_ref, wa_ref, ba_ref, wb_ref, bb_ref, g_ref, be_ref,
              o_ref, *, residual):
    h = x_ref[...] + agg_ref[...]
    a = jnp.maximum(
        jnp.dot(h, wa_ref[...], preferred_element_type=jnp.float32)
        + ba_ref[...], 0.0)
    t = (jnp.dot(a, wb_ref[...], preferred_element_type=jnp.float32)
         + bb_ref[...])
    mean = jnp.mean(t, axis=0, keepdims=True)
    var = jnp.mean(jnp.square(t - mean), axis=0, keepdims=True)
    y = (t - mean) * lax.rsqrt(var + 1e-5) * g_ref[...] + be_ref[...]
    y = jnp.maximum(y, 0.0)
    if residual:
        y = y + x_ref[...]
    o_ref[...] = y


def _mlp(x, agg, wa, ba, wb, bb, g, be, residual):
    body = functools.partial(_mlp_body, residual=residual)
    return pl.pallas_call(
        body,
        out_shape=jax.ShapeDtypeStruct((N_NODES, D), jnp.float32),
    )(x, agg, wa, ba.reshape(1, D), wb, bb.reshape(1, D),
      g.reshape(1, D), be.reshape(1, D))


def kernel(x, edge_index, edge_weight,
           W0a, b0a, W0b, b0b, g0, be0,
           W1a, b1a, W1b, b1b, g1, be1,
           W2a, b2a, W2b, b2b, g2, be2):
    ei = edge_index.astype(jnp.int32)
    pad = E_PAD - N_EDGES
    src = jnp.pad(ei[0], (0, pad))
    dst = jnp.pad(ei[1], (0, pad))
    edges = jnp.stack([src, dst], axis=1).reshape(
        NSUB * NCHUNK, CHUNK, 2).transpose(0, 2, 1)
    w = jnp.pad(edge_weight, (0, pad)).reshape(NSUB * NCHUNK, CHUNK)

    def gin(h, wa, ba, wb, bb, g, be, residual):
        agg = _segment_sum(h, edges, w)
        return _mlp(h, agg, wa, ba, wb, bb, g, be, residual)

    h = gin(x, W0a, b0a, W0b, b0b, g0, be0, False)
    h = gin(h, W1a, b1a, W1b, b1b, g1, be1, True)
    return gin(h, W2a, b2a, W2b, b2b, g2, be2, True)
